# Initial kernel scaffold; baseline (speedup 1.0000x reference)
import jax
import jax.numpy as jnp
from jax.experimental import pallas as pl


def kernel(features, class_labels, concept_labels):
    raise NotImplementedError("write your pallas kernel here")


# speedup vs baseline: 5.7969x; 5.7969x over previous
import jax
import jax.numpy as jnp
from jax.experimental import pallas as pl
from jax.experimental.pallas import tpu as pltpu

TEMP = 0.07
N = 16384
D = 128
P = 1024
BLK = 2048
NBLK = N // BLK


def _stage1_body(feat_ref, lab_ref, fn_ref, protos_ref, counts_ref):
    i = pl.program_id(0)
    f = feat_ref[...]
    ss = jnp.sum(f * f, axis=1, keepdims=True)
    fn = f / jnp.maximum(jnp.sqrt(ss), 1e-12)
    fn_ref[...] = fn
    lab = lab_ref[0, 0, :]
    oh = (lab[None, :] == jax.lax.broadcasted_iota(jnp.int32, (P, BLK), 0)
          ).astype(jnp.float32)
    psum = jax.lax.dot_general(oh, fn, (((1,), (0,)), ((), ())),
                               preferred_element_type=jnp.float32)
    cnt = jnp.sum(oh, axis=1)

    @pl.when(i == 0)
    def _():
        protos_ref[...] = psum
        counts_ref[0, :] = cnt

    @pl.when(i > 0)
    def _():
        protos_ref[...] += psum
        counts_ref[0, :] += cnt


def _stage2_body(fn_ref, protos_ref, counts_ref, lab_ref, loss_ref):
    i = pl.program_id(0)
    fn = fn_ref[...]
    cnt = counts_ref[0, :]
    inv = 1.0 / (TEMP * (cnt + 1e-9))
    neg = jnp.where(cnt > 0.0, 0.0, -1e30)
    logits = jax.lax.dot_general(fn, protos_ref[...], (((1,), (1,)), ((), ())),
                                 preferred_element_type=jnp.float32)
    logits = logits * inv[None, :]
    masked = logits + neg[None, :]
    m = jnp.max(masked, axis=1, keepdims=True)
    lse = m + jnp.log(jnp.sum(jnp.exp(masked - m), axis=1, keepdims=True))
    lab = lab_ref[0, 0, :]
    oh = (lab[:, None] == jax.lax.broadcasted_iota(jnp.int32, (BLK, P), 1)
          ).astype(jnp.float32)
    tgt = jnp.sum(logits * oh, axis=1, keepdims=True)
    part = jnp.sum(lse - tgt, axis=0, keepdims=True)

    @pl.when(i == 0)
    def _():
        loss_ref[...] = part

    @pl.when(i > 0)
    def _():
        loss_ref[...] += part

    @pl.when(i == NBLK - 1)
    def _():
        loss_ref[...] = loss_ref[...] / N


def kernel(features, class_labels, concept_labels):
    del class_labels
    lab3 = concept_labels.reshape(NBLK, 1, BLK)

    fn, protos, counts = pl.pallas_call(
        _stage1_body,
        grid=(NBLK,),
        in_specs=[
            pl.BlockSpec((BLK, D), lambda i: (i, 0)),
            pl.BlockSpec((1, 1, BLK), lambda i: (i, 0, 0)),
        ],
        out_specs=[
            pl.BlockSpec((BLK, D), lambda i: (i, 0)),
            pl.BlockSpec((P, D), lambda i: (0, 0)),
            pl.BlockSpec((1, P), lambda i: (0, 0)),
        ],
        out_shape=[
            jax.ShapeDtypeStruct((N, D), jnp.float32),
            jax.ShapeDtypeStruct((P, D), jnp.float32),
            jax.ShapeDtypeStruct((1, P), jnp.float32),
        ],
    )(features, lab3)

    loss = pl.pallas_call(
        _stage2_body,
        grid=(NBLK,),
        in_specs=[
            pl.BlockSpec((BLK, D), lambda i: (i, 0)),
            pl.BlockSpec((P, D), lambda i: (0, 0)),
            pl.BlockSpec((1, P), lambda i: (0, 0)),
            pl.BlockSpec((1, 1, BLK), lambda i: (i, 0, 0)),
        ],
        out_specs=pl.BlockSpec((1, 1), lambda i: (0, 0)),
        out_shape=jax.ShapeDtypeStruct((1, 1), jnp.float32),
    )(fn, protos, counts, lab3)

    return loss[0, 0]


# speedup vs baseline: 7.8162x; 1.3483x over previous
import jax
import jax.numpy as jnp
from jax.experimental import pallas as pl
from jax.experimental.pallas import tpu as pltpu

TEMP = 0.07
N = 16384
D = 128
DE = 136
P = 1024
BLK = 2048
NBLK = N // BLK


def _stage1_body(feat_ref, lab_ref, fn_ref, pp_ref):
    i = pl.program_id(0)
    f = feat_ref[...]
    ss = jnp.sum(f * f, axis=1, keepdims=True)
    fn = f * jax.lax.rsqrt(jnp.maximum(ss, 1e-24))
    fn_ref[...] = fn
    lab = lab_ref[0, 0, :]
    oh = (lab[None, :] == jax.lax.broadcasted_iota(jnp.int32, (P, BLK), 0)
          ).astype(jnp.float32)
    fn_ext = jnp.concatenate([fn, jnp.ones((BLK, DE - D), jnp.float32)], axis=1)
    psum = jax.lax.dot_general(oh, fn_ext, (((1,), (0,)), ((), ())),
                               preferred_element_type=jnp.float32)

    @pl.when(i == 0)
    def _():
        pp_ref[...] = psum

    @pl.when(i > 0)
    def _():
        pp_ref[...] += psum


def _stage2_body(fn_ref, pp_ref, lab_ref, loss_ref, ps_ref):
    i = pl.program_id(0)

    @pl.when(i == 0)
    def _():
        pp = pp_ref[...]
        protos = pp[:, 0:D]
        cnt = pp[:, D:D + 1]
        inv = 1.0 / (TEMP * (cnt + 1e-9))
        pscale = protos * inv
        bias = jnp.where(cnt > 0.0, 0.0, -1e30)
        ps_ref[...] = jnp.concatenate(
            [pscale, bias, jnp.zeros((P, DE - D - 1), jnp.float32)], axis=1)

    fn = fn_ref[...]
    fn_ext = jnp.concatenate([fn, jnp.ones((BLK, DE - D), jnp.float32)], axis=1)
    z = jax.lax.dot_general(fn_ext, ps_ref[...], (((1,), (1,)), ((), ())),
                            preferred_element_type=jnp.float32)
    s = jnp.sum(jnp.exp(z), axis=1, keepdims=True)
    lab = lab_ref[0, 0, :]
    oh = (lab[:, None] == jax.lax.broadcasted_iota(jnp.int32, (BLK, P), 1)
          ).astype(jnp.float32)
    tgt = jnp.sum(z * oh, axis=1, keepdims=True)
    part = jnp.sum(jnp.log(s) - tgt, axis=0, keepdims=True)

    @pl.when(i == 0)
    def _():
        loss_ref[...] = part

    @pl.when(i > 0)
    def _():
        loss_ref[...] += part

    @pl.when(i == NBLK - 1)
    def _():
        loss_ref[...] = loss_ref[...] / N


def kernel(features, class_labels, concept_labels):
    del class_labels
    lab3 = concept_labels.reshape(NBLK, 1, BLK)

    fn, pp = pl.pallas_call(
        _stage1_body,
        grid=(NBLK,),
        in_specs=[
            pl.BlockSpec((BLK, D), lambda i: (i, 0)),
            pl.BlockSpec((1, 1, BLK), lambda i: (i, 0, 0)),
        ],
        out_specs=[
            pl.BlockSpec((BLK, D), lambda i: (i, 0)),
            pl.BlockSpec((P, DE), lambda i: (0, 0)),
        ],
        out_shape=[
            jax.ShapeDtypeStruct((N, D), jnp.float32),
            jax.ShapeDtypeStruct((P, DE), jnp.float32),
        ],
    )(features, lab3)

    loss = pl.pallas_call(
        _stage2_body,
        grid=(NBLK,),
        in_specs=[
            pl.BlockSpec((BLK, D), lambda i: (i, 0)),
            pl.BlockSpec((P, DE), lambda i: (0, 0)),
            pl.BlockSpec((1, 1, BLK), lambda i: (i, 0, 0)),
        ],
        out_specs=pl.BlockSpec((1, 1), lambda i: (0, 0)),
        out_shape=jax.ShapeDtypeStruct((1, 1), jnp.float32),
        scratch_shapes=[pltpu.VMEM((P, DE), jnp.float32)],
    )(fn, pp, lab3)

    return loss[0, 0]
